# baseline (speedup 1.0000x reference)
import functools

import jax
import jax.numpy as jnp
from jax import lax
from jax.experimental import pallas as pl
from jax.experimental.pallas import tpu as pltpu
from jax.experimental.pallas import tpu_sc as plsc

N = 10000
E = 320000
F = 128
FH = F // 2
RES = 100
NC = 2
NS = 16
NW = NC * NS
EPW = E // NW
EPT = E // NS
CH = 80
NCHT = EPT // CH
RPT = 624
TAIL_OFF = RPT * NS
TAIL = N - TAIL_OFF
ZR = 48
NZ = RPT // ZR


def _sc_mesh():
    return plsc.VectorSubcoreMesh(core_axis_name="c", subcore_axis_name="s")


def _deg_parts(row3, ew):
    @functools.partial(
        pl.kernel,
        out_type=jax.ShapeDtypeStruct((NC * N,), jnp.float32),
        mesh=_sc_mesh(),
        scratch_types=[
            pltpu.VMEM((EPW // CH, CH), jnp.int32),
            pltpu.VMEM((EPW,), jnp.float32),
            pltpu.VMEM((RPT,), jnp.float32),
            pltpu.VMEM_SHARED((N,), jnp.float32),
        ],
    )
    def k(row_h, ew_h, out_h, row_v, ew_v, zbuf, acc):
        c = lax.axis_index("c")
        s = lax.axis_index("s")
        wid = c * NS + s
        base = wid * EPW
        pltpu.sync_copy(row_h.at[wid], row_v)
        pltpu.sync_copy(ew_h.at[pl.ds(base, EPW)], ew_v)
        z16 = jnp.zeros((16,), jnp.float32)
        for i in range(RPT // 16):
            zbuf[pl.ds(i * 16, 16)] = z16
        pltpu.sync_copy(zbuf, acc.at[pl.ds(s * RPT, RPT)])

        @pl.when(s == NS - 1)
        def _():
            pltpu.sync_copy(zbuf.at[pl.ds(0, TAIL)], acc.at[pl.ds(TAIL_OFF, TAIL)])

        plsc.subcore_barrier()

        def chunk(ch, _):
            pltpu.sync_copy(ew_v.at[pl.ds(ch * CH, CH)],
                            acc.at[row_v.at[ch]], add=True)
            return _

        lax.fori_loop(0, EPW // CH, chunk, 0)
        plsc.subcore_barrier()
        pltpu.sync_copy(acc.at[pl.ds(s * RPT, RPT)], zbuf)
        pltpu.sync_copy(zbuf, out_h.at[pl.ds(c * N + s * RPT, RPT)])

        @pl.when(s == NS - 1)
        def _():
            pltpu.sync_copy(acc.at[pl.ds(TAIL_OFF, TAIL)], zbuf.at[pl.ds(0, TAIL)])
            pltpu.sync_copy(zbuf.at[pl.ds(0, TAIL)],
                            out_h.at[pl.ds(c * N + TAIL_OFF, TAIL)])

    return k(row3, ew)


def _w_scaled(row, col, ew, dinv):
    @functools.partial(
        pl.kernel,
        out_type=jax.ShapeDtypeStruct((E,), jnp.float32),
        mesh=_sc_mesh(),
        compiler_params=pltpu.CompilerParams(needs_layout_passes=False),
        scratch_types=[
            pltpu.VMEM((EPW,), jnp.int32),
            pltpu.VMEM((EPW,), jnp.int32),
            pltpu.VMEM((EPW,), jnp.float32),
            pltpu.VMEM((N,), jnp.float32),
            pltpu.VMEM((EPW,), jnp.float32),
        ],
    )
    def k(row_h, col_h, ew_h, dinv_h, out_h, row_v, col_v, ew_v, dinv_v, ws_v):
        c = lax.axis_index("c")
        s = lax.axis_index("s")
        base = (c * NS + s) * EPW
        pltpu.sync_copy(row_h.at[pl.ds(base, EPW)], row_v)
        pltpu.sync_copy(col_h.at[pl.ds(base, EPW)], col_v)
        pltpu.sync_copy(ew_h.at[pl.ds(base, EPW)], ew_v)
        pltpu.sync_copy(dinv_h, dinv_v)

        def step(i, _):
            o = i * 16
            r16 = row_v[pl.ds(o, 16)]
            c16 = col_v[pl.ds(o, 16)]
            w16 = ew_v[pl.ds(o, 16)]
            dr = plsc.load_gather(dinv_v, [r16])
            dc = plsc.load_gather(dinv_v, [c16])
            ws_v[pl.ds(o, 16)] = -(dr * w16 * dc)
            return _

        lax.fori_loop(0, EPW // 16, step, 0)
        pltpu.sync_copy(ws_v, out_h.at[pl.ds(base, EPW)])

    return k(row, col, ew, dinv)


NBUF = 5
NK = NCHT // NBUF


def _lhat_halves(src2, row3l, col2, ws):
    @functools.partial(
        pl.kernel,
        out_type=jax.ShapeDtypeStruct((NC, N, FH), jnp.float32),
        mesh=_sc_mesh(),
        compiler_params=pltpu.CompilerParams(use_tc_tiling_on_sc=False),
        scratch_types=[
            pltpu.VMEM_SHARED((N, FH), jnp.float32),
            pltpu.VMEM((NCHT, CH), jnp.int32),
            pltpu.VMEM((EPT,), jnp.int32),
            pltpu.VMEM((EPT,), jnp.float32),
            pltpu.VMEM((ZR, FH), jnp.float32),
        ] + [pltpu.VMEM((CH, FH), jnp.float32) for _ in range(NBUF)]
          + [pltpu.SemaphoreType.DMA for _ in range(2 * NBUF)],
    )
    def k(src_h, row_h, col_h, ws_h, out_h,
          acc, row_v, col_v, ws_v, zbuf, *bufsem):
        bufs = bufsem[:NBUF]
        gsem = bufsem[NBUF:2 * NBUF]
        ssem = bufsem[2 * NBUF:]
        c = lax.axis_index("c")
        s = lax.axis_index("s")
        base = s * EPT
        pltpu.sync_copy(row_h.at[s], row_v)
        pltpu.sync_copy(col_h.at[pl.ds(c * E + base, EPT)], col_v)
        pltpu.sync_copy(ws_h.at[pl.ds(base, EPT)], ws_v)

        z16 = jnp.zeros((16,), jnp.float32)
        for i in range(ZR):
            for j in range(FH // 16):
                zbuf[i, pl.ds(j * 16, 16)] = z16
        for q in range(NZ):
            pltpu.sync_copy(zbuf, acc.at[pl.ds(s * RPT + q * ZR, ZR)])

        @pl.when(s == NS - 1)
        def _():
            pltpu.sync_copy(zbuf.at[pl.ds(0, TAIL)], acc.at[pl.ds(TAIL_OFF, TAIL)])

        plsc.subcore_barrier()

        def gissue(ch, b):
            pltpu.async_copy(src_h.at[col_v.at[pl.ds(ch * CH, CH)]],
                             bufs[b], gsem[b])

        for b in range(NBUF):
            gissue(b, b)

        def round_(k_, car):
            for b in range(NBUF):
                ch = k_ * NBUF + b
                o = ch * CH
                pltpu.make_async_copy(src_h.at[pl.ds(0, CH)],
                                      bufs[b], gsem[b]).wait()

                def scale(g, _, b=b, o=o):
                    wvec = ws_v[pl.ds(o + g * 16, 16)]
                    for i in range(16):
                        w = wvec[i]
                        r = g * 16 + i
                        for j in range(FH // 16):
                            bufs[b][r, pl.ds(j * 16, 16)] = (
                                bufs[b][r, pl.ds(j * 16, 16)] * w)
                    return _

                lax.fori_loop(0, CH // 16, scale, 0)
                pltpu.async_copy(bufs[b], acc.at[row_v.at[ch]], ssem[b],
                                 add=True)

            @pl.when(k_ < NK - 1)
            def _prefetch():
                for b in range(NBUF):
                    pltpu.make_async_copy(src_h.at[pl.ds(0, CH)],
                                          bufs[b], ssem[b]).wait()
                    gissue((k_ + 1) * NBUF + b, b)

            return car

        lax.fori_loop(0, NK, round_, 0)
        for b in range(NBUF):
            pltpu.make_async_copy(src_h.at[pl.ds(0, CH)],
                                  bufs[b], ssem[b]).wait()
        plsc.subcore_barrier()
        for q in range(NZ):
            pltpu.sync_copy(acc.at[pl.ds(s * RPT + q * ZR, ZR)], zbuf)
            pltpu.sync_copy(zbuf, out_h.at[c, pl.ds(s * RPT + q * ZR, ZR)])

        @pl.when(s == NS - 1)
        def _():
            pltpu.sync_copy(acc.at[pl.ds(TAIL_OFF, TAIL)], zbuf.at[pl.ds(0, TAIL)])
            pltpu.sync_copy(zbuf.at[pl.ds(0, TAIL)],
                            out_h.at[c, pl.ds(TAIL_OFF, TAIL)])

    return k(src2, row3l, col2, ws)


BN = 400


def _split(x):
    def body(x_ref, o_ref):
        o_ref[0] = x_ref[:, :FH]
        o_ref[1] = x_ref[:, FH:]

    return pl.pallas_call(
        body,
        grid=(N // BN,),
        in_specs=[pl.BlockSpec((BN, F), lambda i: (i, 0))],
        out_specs=pl.BlockSpec((2, BN, FH), lambda i: (0, i, 0)),
        out_shape=jax.ShapeDtypeStruct((2, N, FH), jnp.float32),
    )(x)


def _mix(src, t1, p2, W, b, relu, split_out):
    def body(s0, s1, t0, t1r, p0, p1, w_ref, b_ref, o_ref):
        xb = jnp.concatenate([s0[0], s1[0]], axis=1)
        tb = jnp.concatenate([t0[0], t1r[0]], axis=1)
        lb = jnp.concatenate([p0[0], p1[0]], axis=1)
        acc = jnp.dot(xb, w_ref[0] - w_ref[2], preferred_element_type=jnp.float32)
        acc += jnp.dot(tb, w_ref[1], preferred_element_type=jnp.float32)
        acc += jnp.dot(lb, w_ref[2] * 2.0, preferred_element_type=jnp.float32)
        acc += b_ref[...]
        if relu:
            acc = jnp.maximum(acc, 0.0)
        if split_out:
            o_ref[0] = acc[:, :FH]
            o_ref[1] = acc[:, FH:]
        else:
            o_ref[...] = acc

    half = lambda h: pl.BlockSpec((1, BN, FH), lambda i, _h=h: (_h, i, 0))
    if split_out:
        out_spec = pl.BlockSpec((2, BN, FH), lambda i: (0, i, 0))
        out_shape = jax.ShapeDtypeStruct((2, N, FH), jnp.float32)
    else:
        out_spec = pl.BlockSpec((BN, F), lambda i: (i, 0))
        out_shape = jax.ShapeDtypeStruct((N, F), jnp.float32)
    return pl.pallas_call(
        body,
        grid=(N // BN,),
        in_specs=[half(0), half(1), half(0), half(1), half(0), half(1),
                  pl.BlockSpec((3, F, F), lambda i: (0, 0, 0)),
                  pl.BlockSpec((1, F), lambda i: (0, 0))],
        out_specs=out_spec,
        out_shape=out_shape,
    )(src, src, t1, t1, p2, p2, W, b.reshape(1, F))


def _fc_head(h, fc1_w, fc1_b, fc2_w, fc2_b, fc3_w, fc3_b):
    def body(h_ref, w1_ref, b1_ref, w2_ref, b2_ref, w3_ref, b3_ref, o_ref):
        g = jnp.dot(h_ref[...], w1_ref[...], preferred_element_type=jnp.float32)
        g += b1_ref[...]
        g = jnp.dot(g, w2_ref[...], preferred_element_type=jnp.float32)
        g += b2_ref[...]
        g = jnp.dot(g, w3_ref[...], preferred_element_type=jnp.float32)
        g += b3_ref[...]
        o_ref[...] = g

    n_cls = fc3_w.shape[1]
    return pl.pallas_call(
        body,
        out_shape=jax.ShapeDtypeStruct((RES, n_cls), jnp.float32),
    )(h, fc1_w, fc1_b.reshape(1, -1), fc2_w, fc2_b.reshape(1, -1),
      fc3_w, fc3_b.reshape(1, -1))


def kernel(x, edge_index, edge_weight, W1, b1, W2, b2,
           fc1_w, fc1_b, fc2_w, fc2_b, fc3_w, fc3_b):
    row = edge_index[0]
    col = edge_index[1]
    row3 = row.reshape(NW, EPW // CH, CH)
    row3l = row.reshape(NS, NCHT, CH)
    col2 = jnp.concatenate([col, col + N])

    deg = _deg_parts(row3, edge_weight).reshape(NC, N).sum(axis=0)
    dinv = jnp.where(deg > 0, jax.lax.rsqrt(jnp.where(deg > 0, deg, 1.0)), 0.0)
    ws = _w_scaled(row, col, edge_weight, dinv)

    def layer(src_split, W, b, relu, split_out):
        p1 = _lhat_halves(src_split.reshape(NC * N, FH), row3l, col2, ws)
        p2 = _lhat_halves(p1.reshape(NC * N, FH), row3l, col2, ws)
        return _mix(src_split, p1, p2, W, b, relu, split_out)

    xh = _split(x)
    h = layer(xh, W1, b1, True, True)
    h2 = layer(h, W2, b2, False, False)
    return _fc_head(h2.reshape(RES, RES * F),
                    fc1_w, fc1_b, fc2_w, fc2_b, fc3_w, fc3_b)


# speedup vs baseline: 1.7141x; 1.7141x over previous
import functools

import jax
import jax.numpy as jnp
from jax import lax
from jax.experimental import pallas as pl
from jax.experimental.pallas import tpu as pltpu
from jax.experimental.pallas import tpu_sc as plsc

N = 10000
E = 320000
F = 128
FH = F // 2
RES = 100
NC = 2
NS = 16
NW = NC * NS
EPW = E // NW
EPT = E // NS
CH = 80
NCHT = EPT // CH
RPT = 624
TAIL_OFF = RPT * NS
TAIL = N - TAIL_OFF
ZR = 48
NZ = RPT // ZR


def _sc_mesh():
    return plsc.VectorSubcoreMesh(core_axis_name="c", subcore_axis_name="s")


def _deg_parts(row3, ew):
    @functools.partial(
        pl.kernel,
        out_type=jax.ShapeDtypeStruct((NC * N,), jnp.float32),
        mesh=_sc_mesh(),
        scratch_types=[
            pltpu.VMEM((EPW // CH, CH), jnp.int32),
            pltpu.VMEM((EPW,), jnp.float32),
            pltpu.VMEM((RPT,), jnp.float32),
            pltpu.VMEM_SHARED((N,), jnp.float32),
        ],
    )
    def k(row_h, ew_h, out_h, row_v, ew_v, zbuf, acc):
        c = lax.axis_index("c")
        s = lax.axis_index("s")
        wid = c * NS + s
        base = wid * EPW
        pltpu.sync_copy(row_h.at[wid], row_v)
        pltpu.sync_copy(ew_h.at[pl.ds(base, EPW)], ew_v)
        z16 = jnp.zeros((16,), jnp.float32)
        for i in range(RPT // 16):
            zbuf[pl.ds(i * 16, 16)] = z16
        pltpu.sync_copy(zbuf, acc.at[pl.ds(s * RPT, RPT)])

        @pl.when(s == NS - 1)
        def _():
            pltpu.sync_copy(zbuf.at[pl.ds(0, TAIL)], acc.at[pl.ds(TAIL_OFF, TAIL)])

        plsc.subcore_barrier()

        def chunk(ch, _):
            pltpu.sync_copy(ew_v.at[pl.ds(ch * CH, CH)],
                            acc.at[row_v.at[ch]], add=True)
            return _

        lax.fori_loop(0, EPW // CH, chunk, 0)
        plsc.subcore_barrier()
        pltpu.sync_copy(acc.at[pl.ds(s * RPT, RPT)], zbuf)
        pltpu.sync_copy(zbuf, out_h.at[pl.ds(c * N + s * RPT, RPT)])

        @pl.when(s == NS - 1)
        def _():
            pltpu.sync_copy(acc.at[pl.ds(TAIL_OFF, TAIL)], zbuf.at[pl.ds(0, TAIL)])
            pltpu.sync_copy(zbuf.at[pl.ds(0, TAIL)],
                            out_h.at[pl.ds(c * N + TAIL_OFF, TAIL)])

    return k(row3, ew)


def _w_scaled(row, col, ew, dinv):
    @functools.partial(
        pl.kernel,
        out_type=jax.ShapeDtypeStruct((E,), jnp.float32),
        mesh=_sc_mesh(),
        compiler_params=pltpu.CompilerParams(needs_layout_passes=False),
        scratch_types=[
            pltpu.VMEM((EPW,), jnp.int32),
            pltpu.VMEM((EPW,), jnp.int32),
            pltpu.VMEM((EPW,), jnp.float32),
            pltpu.VMEM((N,), jnp.float32),
            pltpu.VMEM((EPW,), jnp.float32),
        ],
    )
    def k(row_h, col_h, ew_h, dinv_h, out_h, row_v, col_v, ew_v, dinv_v, ws_v):
        c = lax.axis_index("c")
        s = lax.axis_index("s")
        base = (c * NS + s) * EPW
        pltpu.sync_copy(row_h.at[pl.ds(base, EPW)], row_v)
        pltpu.sync_copy(col_h.at[pl.ds(base, EPW)], col_v)
        pltpu.sync_copy(ew_h.at[pl.ds(base, EPW)], ew_v)
        pltpu.sync_copy(dinv_h, dinv_v)

        def step(i, _):
            o = i * 16
            r16 = row_v[pl.ds(o, 16)]
            c16 = col_v[pl.ds(o, 16)]
            w16 = ew_v[pl.ds(o, 16)]
            dr = plsc.load_gather(dinv_v, [r16])
            dc = plsc.load_gather(dinv_v, [c16])
            ws_v[pl.ds(o, 16)] = -(dr * w16 * dc)
            return _

        lax.fori_loop(0, EPW // 16, step, 0)
        pltpu.sync_copy(ws_v, out_h.at[pl.ds(base, EPW)])

    return k(row, col, ew, dinv)


NBUF = 5
NK = NCHT // NBUF


def _lhat_halves(src2, row3l, col2, ws):
    @functools.partial(
        pl.kernel,
        out_type=jax.ShapeDtypeStruct((NC, N, FH), jnp.float32),
        mesh=_sc_mesh(),
        compiler_params=pltpu.CompilerParams(use_tc_tiling_on_sc=False),
        scratch_types=[
            pltpu.VMEM_SHARED((N, FH), jnp.float32),
            pltpu.VMEM((NCHT, CH), jnp.int32),
            pltpu.VMEM((EPT,), jnp.int32),
            pltpu.VMEM((EPT,), jnp.float32),
            pltpu.VMEM((ZR, FH), jnp.float32),
        ] + [pltpu.VMEM((CH, FH), jnp.float32) for _ in range(NBUF)]
          + [pltpu.SemaphoreType.DMA for _ in range(2 * NBUF)],
    )
    def k(src_h, row_h, col_h, ws_h, out_h,
          acc, row_v, col_v, ws_v, zbuf, *bufsem):
        bufs = bufsem[:NBUF]
        gsem = bufsem[NBUF:2 * NBUF]
        ssem = bufsem[2 * NBUF:]
        c = lax.axis_index("c")
        s = lax.axis_index("s")
        base = s * EPT
        pltpu.sync_copy(row_h.at[s], row_v)
        pltpu.sync_copy(col_h.at[pl.ds(c * E + base, EPT)], col_v)
        pltpu.sync_copy(ws_h.at[pl.ds(base, EPT)], ws_v)

        z16 = jnp.zeros((16,), jnp.float32)
        for i in range(ZR):
            for j in range(FH // 16):
                zbuf[i, pl.ds(j * 16, 16)] = z16
        for q in range(NZ):
            pltpu.sync_copy(zbuf, acc.at[pl.ds(s * RPT + q * ZR, ZR)])

        @pl.when(s == NS - 1)
        def _():
            pltpu.sync_copy(zbuf.at[pl.ds(0, TAIL)], acc.at[pl.ds(TAIL_OFF, TAIL)])

        plsc.subcore_barrier()

        def gissue(ch, b):
            pltpu.async_copy(src_h.at[col_v.at[pl.ds(ch * CH, CH)]],
                             bufs[b], gsem[b])

        for b in range(NBUF):
            gissue(b, b)

        def round_(k_, car):
            for b in range(NBUF):
                ch = k_ * NBUF + b
                o = ch * CH
                pltpu.make_async_copy(src_h.at[pl.ds(0, CH)],
                                      bufs[b], gsem[b]).wait()

                for g in range(CH // 16):
                    wvec = ws_v[pl.ds(o + g * 16, 16)]
                    for i in range(16):
                        w = wvec[i]
                        r = g * 16 + i
                        for j in range(FH // 16):
                            bufs[b][r, pl.ds(j * 16, 16)] = (
                                bufs[b][r, pl.ds(j * 16, 16)] * w)
                pltpu.async_copy(bufs[b], acc.at[row_v.at[ch]], ssem[b],
                                 add=True)

            @pl.when(k_ < NK - 1)
            def _prefetch():
                for b in range(NBUF):
                    pltpu.make_async_copy(src_h.at[pl.ds(0, CH)],
                                          bufs[b], ssem[b]).wait()
                    gissue((k_ + 1) * NBUF + b, b)

            return car

        lax.fori_loop(0, NK, round_, 0)
        for b in range(NBUF):
            pltpu.make_async_copy(src_h.at[pl.ds(0, CH)],
                                  bufs[b], ssem[b]).wait()
        plsc.subcore_barrier()
        for q in range(NZ):
            pltpu.sync_copy(acc.at[pl.ds(s * RPT + q * ZR, ZR)], zbuf)
            pltpu.sync_copy(zbuf, out_h.at[c, pl.ds(s * RPT + q * ZR, ZR)])

        @pl.when(s == NS - 1)
        def _():
            pltpu.sync_copy(acc.at[pl.ds(TAIL_OFF, TAIL)], zbuf.at[pl.ds(0, TAIL)])
            pltpu.sync_copy(zbuf.at[pl.ds(0, TAIL)],
                            out_h.at[c, pl.ds(TAIL_OFF, TAIL)])

    return k(src2, row3l, col2, ws)


BN = 400


def _split(x):
    def body(x_ref, o_ref):
        o_ref[0] = x_ref[:, :FH]
        o_ref[1] = x_ref[:, FH:]

    return pl.pallas_call(
        body,
        grid=(N // BN,),
        in_specs=[pl.BlockSpec((BN, F), lambda i: (i, 0))],
        out_specs=pl.BlockSpec((2, BN, FH), lambda i: (0, i, 0)),
        out_shape=jax.ShapeDtypeStruct((2, N, FH), jnp.float32),
    )(x)


def _mix(src, t1, p2, W, b, relu, split_out):
    def body(s0, s1, t0, t1r, p0, p1, w_ref, b_ref, o_ref):
        xb = jnp.concatenate([s0[0], s1[0]], axis=1)
        tb = jnp.concatenate([t0[0], t1r[0]], axis=1)
        lb = jnp.concatenate([p0[0], p1[0]], axis=1)
        acc = jnp.dot(xb, w_ref[0] - w_ref[2], preferred_element_type=jnp.float32)
        acc += jnp.dot(tb, w_ref[1], preferred_element_type=jnp.float32)
        acc += jnp.dot(lb, w_ref[2] * 2.0, preferred_element_type=jnp.float32)
        acc += b_ref[...]
        if relu:
            acc = jnp.maximum(acc, 0.0)
        if split_out:
            o_ref[0] = acc[:, :FH]
            o_ref[1] = acc[:, FH:]
        else:
            o_ref[...] = acc

    half = lambda h: pl.BlockSpec((1, BN, FH), lambda i, _h=h: (_h, i, 0))
    if split_out:
        out_spec = pl.BlockSpec((2, BN, FH), lambda i: (0, i, 0))
        out_shape = jax.ShapeDtypeStruct((2, N, FH), jnp.float32)
    else:
        out_spec = pl.BlockSpec((BN, F), lambda i: (i, 0))
        out_shape = jax.ShapeDtypeStruct((N, F), jnp.float32)
    return pl.pallas_call(
        body,
        grid=(N // BN,),
        in_specs=[half(0), half(1), half(0), half(1), half(0), half(1),
                  pl.BlockSpec((3, F, F), lambda i: (0, 0, 0)),
                  pl.BlockSpec((1, F), lambda i: (0, 0))],
        out_specs=out_spec,
        out_shape=out_shape,
    )(src, src, t1, t1, p2, p2, W, b.reshape(1, F))


def _fc_head(h, fc1_w, fc1_b, fc2_w, fc2_b, fc3_w, fc3_b):
    def body(h_ref, w1_ref, b1_ref, w2_ref, b2_ref, w3_ref, b3_ref, o_ref):
        g = jnp.dot(h_ref[...], w1_ref[...], preferred_element_type=jnp.float32)
        g += b1_ref[...]
        g = jnp.dot(g, w2_ref[...], preferred_element_type=jnp.float32)
        g += b2_ref[...]
        g = jnp.dot(g, w3_ref[...], preferred_element_type=jnp.float32)
        g += b3_ref[...]
        o_ref[...] = g

    n_cls = fc3_w.shape[1]
    return pl.pallas_call(
        body,
        out_shape=jax.ShapeDtypeStruct((RES, n_cls), jnp.float32),
    )(h, fc1_w, fc1_b.reshape(1, -1), fc2_w, fc2_b.reshape(1, -1),
      fc3_w, fc3_b.reshape(1, -1))


def kernel(x, edge_index, edge_weight, W1, b1, W2, b2,
           fc1_w, fc1_b, fc2_w, fc2_b, fc3_w, fc3_b):
    row = edge_index[0]
    col = edge_index[1]
    row3 = row.reshape(NW, EPW // CH, CH)
    row3l = row.reshape(NS, NCHT, CH)
    col2 = jnp.concatenate([col, col + N])

    deg = _deg_parts(row3, edge_weight).reshape(NC, N).sum(axis=0)
    dinv = jnp.where(deg > 0, jax.lax.rsqrt(jnp.where(deg > 0, deg, 1.0)), 0.0)
    ws = _w_scaled(row, col, edge_weight, dinv)

    def layer(src_split, W, b, relu, split_out):
        p1 = _lhat_halves(src_split.reshape(NC * N, FH), row3l, col2, ws)
        p2 = _lhat_halves(p1.reshape(NC * N, FH), row3l, col2, ws)
        return _mix(src_split, p1, p2, W, b, relu, split_out)

    xh = _split(x)
    h = layer(xh, W1, b1, True, True)
    h2 = layer(h, W2, b2, False, False)
    return _fc_head(h2.reshape(RES, RES * F),
                    fc1_w, fc1_b, fc2_w, fc2_b, fc3_w, fc3_b)


# speedup vs baseline: 2.2769x; 1.3283x over previous
import functools

import jax
import jax.numpy as jnp
from jax import lax
from jax.experimental import pallas as pl
from jax.experimental.pallas import tpu as pltpu
from jax.experimental.pallas import tpu_sc as plsc

N = 10000
E = 320000
F = 128
FH = F // 2
RES = 100
NC = 2
NS = 16
NW = NC * NS
EPW = E // NW
EPT = E // NS
CH = 80
NCHT = EPT // CH
RPT = 624
TAIL_OFF = RPT * NS
TAIL = N - TAIL_OFF
ZR = 48
NZ = RPT // ZR


def _sc_mesh():
    return plsc.VectorSubcoreMesh(core_axis_name="c", subcore_axis_name="s")


def _deg_parts(row3, ew):
    @functools.partial(
        pl.kernel,
        out_type=jax.ShapeDtypeStruct((NC * N,), jnp.float32),
        mesh=_sc_mesh(),
        scratch_types=[
            pltpu.VMEM((EPW // CH, CH), jnp.int32),
            pltpu.VMEM((EPW,), jnp.float32),
            pltpu.VMEM((RPT,), jnp.float32),
            pltpu.VMEM_SHARED((N,), jnp.float32),
        ],
    )
    def k(row_h, ew_h, out_h, row_v, ew_v, zbuf, acc):
        c = lax.axis_index("c")
        s = lax.axis_index("s")
        wid = c * NS + s
        base = wid * EPW
        pltpu.sync_copy(row_h.at[wid], row_v)
        pltpu.sync_copy(ew_h.at[pl.ds(base, EPW)], ew_v)
        z16 = jnp.zeros((16,), jnp.float32)
        for i in range(RPT // 16):
            zbuf[pl.ds(i * 16, 16)] = z16
        pltpu.sync_copy(zbuf, acc.at[pl.ds(s * RPT, RPT)])

        @pl.when(s == NS - 1)
        def _():
            pltpu.sync_copy(zbuf.at[pl.ds(0, TAIL)], acc.at[pl.ds(TAIL_OFF, TAIL)])

        plsc.subcore_barrier()

        def chunk(ch, _):
            pltpu.sync_copy(ew_v.at[pl.ds(ch * CH, CH)],
                            acc.at[row_v.at[ch]], add=True)
            return _

        lax.fori_loop(0, EPW // CH, chunk, 0)
        plsc.subcore_barrier()
        pltpu.sync_copy(acc.at[pl.ds(s * RPT, RPT)], zbuf)
        pltpu.sync_copy(zbuf, out_h.at[pl.ds(c * N + s * RPT, RPT)])

        @pl.when(s == NS - 1)
        def _():
            pltpu.sync_copy(acc.at[pl.ds(TAIL_OFF, TAIL)], zbuf.at[pl.ds(0, TAIL)])
            pltpu.sync_copy(zbuf.at[pl.ds(0, TAIL)],
                            out_h.at[pl.ds(c * N + TAIL_OFF, TAIL)])

    return k(row3, ew)


def _w_scaled(row, col, ew, dinv):
    @functools.partial(
        pl.kernel,
        out_type=jax.ShapeDtypeStruct((E,), jnp.float32),
        mesh=_sc_mesh(),
        compiler_params=pltpu.CompilerParams(needs_layout_passes=False),
        scratch_types=[
            pltpu.VMEM((EPW,), jnp.int32),
            pltpu.VMEM((EPW,), jnp.int32),
            pltpu.VMEM((EPW,), jnp.float32),
            pltpu.VMEM((N,), jnp.float32),
            pltpu.VMEM((EPW,), jnp.float32),
        ],
    )
    def k(row_h, col_h, ew_h, dinv_h, out_h, row_v, col_v, ew_v, dinv_v, ws_v):
        c = lax.axis_index("c")
        s = lax.axis_index("s")
        base = (c * NS + s) * EPW
        pltpu.sync_copy(row_h.at[pl.ds(base, EPW)], row_v)
        pltpu.sync_copy(col_h.at[pl.ds(base, EPW)], col_v)
        pltpu.sync_copy(ew_h.at[pl.ds(base, EPW)], ew_v)
        pltpu.sync_copy(dinv_h, dinv_v)

        def step(i, _):
            o = i * 16
            r16 = row_v[pl.ds(o, 16)]
            c16 = col_v[pl.ds(o, 16)]
            w16 = ew_v[pl.ds(o, 16)]
            dr = plsc.load_gather(dinv_v, [r16])
            dc = plsc.load_gather(dinv_v, [c16])
            ws_v[pl.ds(o, 16)] = -(dr * w16 * dc)
            return _

        lax.fori_loop(0, EPW // 16, step, 0)
        pltpu.sync_copy(ws_v, out_h.at[pl.ds(base, EPW)])

    return k(row, col, ew, dinv)


NBUF = 5
NK = NCHT // NBUF


def _lhat_halves(src2, row3l, col2, ws):
    @functools.partial(
        pl.kernel,
        out_type=jax.ShapeDtypeStruct((NC, N, FH), jnp.float32),
        mesh=_sc_mesh(),
        compiler_params=pltpu.CompilerParams(use_tc_tiling_on_sc=False),
        scratch_types=[
            pltpu.VMEM_SHARED((N, FH), jnp.float32),
            pltpu.VMEM((NCHT, CH), jnp.int32),
            pltpu.VMEM((EPT,), jnp.int32),
            pltpu.VMEM((EPT,), jnp.float32),
            pltpu.VMEM((ZR, FH), jnp.float32),
        ] + [pltpu.VMEM((CH, FH), jnp.float32) for _ in range(NBUF)]
          + [pltpu.SemaphoreType.DMA for _ in range(2 * NBUF)],
    )
    def k(src_h, row_h, col_h, ws_h, out_h,
          acc, row_v, col_v, ws_v, zbuf, *bufsem):
        bufs = bufsem[:NBUF]
        gsem = bufsem[NBUF:2 * NBUF]
        ssem = bufsem[2 * NBUF:]
        c = lax.axis_index("c")
        s = lax.axis_index("s")
        base = s * EPT
        pltpu.sync_copy(row_h.at[s], row_v)
        pltpu.sync_copy(col_h.at[pl.ds(c * E + base, EPT)], col_v)
        pltpu.sync_copy(ws_h.at[pl.ds(base, EPT)], ws_v)

        z16 = jnp.zeros((16,), jnp.float32)
        for i in range(ZR):
            for j in range(FH // 16):
                zbuf[i, pl.ds(j * 16, 16)] = z16
        for q in range(NZ):
            pltpu.sync_copy(zbuf, acc.at[pl.ds(s * RPT + q * ZR, ZR)])

        @pl.when(s == NS - 1)
        def _():
            pltpu.sync_copy(zbuf.at[pl.ds(0, TAIL)], acc.at[pl.ds(TAIL_OFF, TAIL)])

        plsc.subcore_barrier()

        def gissue(ch, b):
            pltpu.async_copy(src_h.at[col_v.at[pl.ds(ch * CH, CH)]],
                             bufs[b], gsem[b])

        def gdrain(b):
            pltpu.make_async_copy(src_h.at[pl.ds(0, CH)],
                                  bufs[b], gsem[b]).wait()

        def sdrain(b):
            pltpu.make_async_copy(src_h.at[pl.ds(0, CH)],
                                  bufs[b], ssem[b]).wait()

        PRE = 3

        for b in range(PRE):
            gissue(b, b)

        def round_(k_, car):
            for j in range(NBUF):
                ch = k_ * NBUF + j
                o = ch * CH
                gdrain(j)
                for g in range(CH // 16):
                    wvec = ws_v[pl.ds(o + g * 16, 16)]
                    for i in range(16):
                        w = wvec[i]
                        r = g * 16 + i
                        for jj in range(FH // 16):
                            bufs[j][r, pl.ds(jj * 16, 16)] = (
                                bufs[j][r, pl.ds(jj * 16, 16)] * w)
                pltpu.async_copy(bufs[j], acc.at[row_v.at[ch]], ssem[j],
                                 add=True)
                tb = (j + PRE) % NBUF
                if j < NBUF - PRE:
                    @pl.when(k_ >= 1)
                    def _w(tb=tb):
                        sdrain(tb)

                    gissue(ch + PRE, tb)
                else:
                    @pl.when(k_ < NK - 1)
                    def _wg(tb=tb, ch=ch):
                        sdrain(tb)
                        gissue(ch + PRE, tb)

            return car

        lax.fori_loop(0, NK, round_, 0)
        for b in range(NBUF):
            sdrain(b)
        plsc.subcore_barrier()
        for q in range(NZ):
            pltpu.sync_copy(acc.at[pl.ds(s * RPT + q * ZR, ZR)], zbuf)
            pltpu.sync_copy(zbuf, out_h.at[c, pl.ds(s * RPT + q * ZR, ZR)])

        @pl.when(s == NS - 1)
        def _():
            pltpu.sync_copy(acc.at[pl.ds(TAIL_OFF, TAIL)], zbuf.at[pl.ds(0, TAIL)])
            pltpu.sync_copy(zbuf.at[pl.ds(0, TAIL)],
                            out_h.at[c, pl.ds(TAIL_OFF, TAIL)])

    return k(src2, row3l, col2, ws)


BN = 400


def _split(x):
    def body(x_ref, o_ref):
        o_ref[0] = x_ref[:, :FH]
        o_ref[1] = x_ref[:, FH:]

    return pl.pallas_call(
        body,
        grid=(N // BN,),
        in_specs=[pl.BlockSpec((BN, F), lambda i: (i, 0))],
        out_specs=pl.BlockSpec((2, BN, FH), lambda i: (0, i, 0)),
        out_shape=jax.ShapeDtypeStruct((2, N, FH), jnp.float32),
    )(x)


def _mix(src, t1, p2, W, b, relu, split_out):
    def body(s0, s1, t0, t1r, p0, p1, w_ref, b_ref, o_ref):
        xb = jnp.concatenate([s0[0], s1[0]], axis=1)
        tb = jnp.concatenate([t0[0], t1r[0]], axis=1)
        lb = jnp.concatenate([p0[0], p1[0]], axis=1)
        acc = jnp.dot(xb, w_ref[0] - w_ref[2], preferred_element_type=jnp.float32)
        acc += jnp.dot(tb, w_ref[1], preferred_element_type=jnp.float32)
        acc += jnp.dot(lb, w_ref[2] * 2.0, preferred_element_type=jnp.float32)
        acc += b_ref[...]
        if relu:
            acc = jnp.maximum(acc, 0.0)
        if split_out:
            o_ref[0] = acc[:, :FH]
            o_ref[1] = acc[:, FH:]
        else:
            o_ref[...] = acc

    half = lambda h: pl.BlockSpec((1, BN, FH), lambda i, _h=h: (_h, i, 0))
    if split_out:
        out_spec = pl.BlockSpec((2, BN, FH), lambda i: (0, i, 0))
        out_shape = jax.ShapeDtypeStruct((2, N, FH), jnp.float32)
    else:
        out_spec = pl.BlockSpec((BN, F), lambda i: (i, 0))
        out_shape = jax.ShapeDtypeStruct((N, F), jnp.float32)
    return pl.pallas_call(
        body,
        grid=(N // BN,),
        in_specs=[half(0), half(1), half(0), half(1), half(0), half(1),
                  pl.BlockSpec((3, F, F), lambda i: (0, 0, 0)),
                  pl.BlockSpec((1, F), lambda i: (0, 0))],
        out_specs=out_spec,
        out_shape=out_shape,
    )(src, src, t1, t1, p2, p2, W, b.reshape(1, F))


def _fc_head(h, fc1_w, fc1_b, fc2_w, fc2_b, fc3_w, fc3_b):
    def body(h_ref, w1_ref, b1_ref, w2_ref, b2_ref, w3_ref, b3_ref, o_ref):
        g = jnp.dot(h_ref[...], w1_ref[...], preferred_element_type=jnp.float32)
        g += b1_ref[...]
        g = jnp.dot(g, w2_ref[...], preferred_element_type=jnp.float32)
        g += b2_ref[...]
        g = jnp.dot(g, w3_ref[...], preferred_element_type=jnp.float32)
        g += b3_ref[...]
        o_ref[...] = g

    n_cls = fc3_w.shape[1]
    return pl.pallas_call(
        body,
        out_shape=jax.ShapeDtypeStruct((RES, n_cls), jnp.float32),
    )(h, fc1_w, fc1_b.reshape(1, -1), fc2_w, fc2_b.reshape(1, -1),
      fc3_w, fc3_b.reshape(1, -1))


def kernel(x, edge_index, edge_weight, W1, b1, W2, b2,
           fc1_w, fc1_b, fc2_w, fc2_b, fc3_w, fc3_b):
    row = edge_index[0]
    col = edge_index[1]
    row3 = row.reshape(NW, EPW // CH, CH)
    row3l = row.reshape(NS, NCHT, CH)
    col2 = jnp.concatenate([col, col + N])

    deg = _deg_parts(row3, edge_weight).reshape(NC, N).sum(axis=0)
    dinv = jnp.where(deg > 0, jax.lax.rsqrt(jnp.where(deg > 0, deg, 1.0)), 0.0)
    ws = _w_scaled(row, col, edge_weight, dinv)

    def layer(src_split, W, b, relu, split_out):
        p1 = _lhat_halves(src_split.reshape(NC * N, FH), row3l, col2, ws)
        p2 = _lhat_halves(p1.reshape(NC * N, FH), row3l, col2, ws)
        return _mix(src_split, p1, p2, W, b, relu, split_out)

    xh = _split(x)
    h = layer(xh, W1, b1, True, True)
    h2 = layer(h, W2, b2, False, False)
    return _fc_head(h2.reshape(RES, RES * F),
                    fc1_w, fc1_b, fc2_w, fc2_b, fc3_w, fc3_b)


# speedup vs baseline: 2.3423x; 1.0287x over previous
import functools

import jax
import jax.numpy as jnp
from jax import lax
from jax.experimental import pallas as pl
from jax.experimental.pallas import tpu as pltpu
from jax.experimental.pallas import tpu_sc as plsc

N = 10000
E = 320000
F = 128
FH = F // 2
RES = 100
NC = 2
NS = 16
NW = NC * NS
EPW = E // NW
EPT = E // NS
CH = 80
NCHT = EPT // CH
RPT = 624
TAIL_OFF = RPT * NS
TAIL = N - TAIL_OFF
ZR = 48
NZ = RPT // ZR


def _sc_mesh():
    return plsc.VectorSubcoreMesh(core_axis_name="c", subcore_axis_name="s")


def _deg_parts(row3, ew):
    @functools.partial(
        pl.kernel,
        out_type=jax.ShapeDtypeStruct((NC * N,), jnp.float32),
        mesh=_sc_mesh(),
        scratch_types=[
            pltpu.VMEM((EPW // CH, CH), jnp.int32),
            pltpu.VMEM((EPW,), jnp.float32),
            pltpu.VMEM((RPT,), jnp.float32),
            pltpu.VMEM_SHARED((N,), jnp.float32),
        ],
    )
    def k(row_h, ew_h, out_h, row_v, ew_v, zbuf, acc):
        c = lax.axis_index("c")
        s = lax.axis_index("s")
        wid = c * NS + s
        base = wid * EPW
        pltpu.sync_copy(row_h.at[wid], row_v)
        pltpu.sync_copy(ew_h.at[pl.ds(base, EPW)], ew_v)
        z16 = jnp.zeros((16,), jnp.float32)
        for i in range(RPT // 16):
            zbuf[pl.ds(i * 16, 16)] = z16
        pltpu.sync_copy(zbuf, acc.at[pl.ds(s * RPT, RPT)])

        @pl.when(s == NS - 1)
        def _():
            pltpu.sync_copy(zbuf.at[pl.ds(0, TAIL)], acc.at[pl.ds(TAIL_OFF, TAIL)])

        plsc.subcore_barrier()

        def chunk(ch, _):
            pltpu.sync_copy(ew_v.at[pl.ds(ch * CH, CH)],
                            acc.at[row_v.at[ch]], add=True)
            return _

        lax.fori_loop(0, EPW // CH, chunk, 0)
        plsc.subcore_barrier()
        pltpu.sync_copy(acc.at[pl.ds(s * RPT, RPT)], zbuf)
        pltpu.sync_copy(zbuf, out_h.at[pl.ds(c * N + s * RPT, RPT)])

        @pl.when(s == NS - 1)
        def _():
            pltpu.sync_copy(acc.at[pl.ds(TAIL_OFF, TAIL)], zbuf.at[pl.ds(0, TAIL)])
            pltpu.sync_copy(zbuf.at[pl.ds(0, TAIL)],
                            out_h.at[pl.ds(c * N + TAIL_OFF, TAIL)])

    return k(row3, ew)


def _w_scaled(row, col, ew, dinv):
    @functools.partial(
        pl.kernel,
        out_type=jax.ShapeDtypeStruct((E,), jnp.float32),
        mesh=_sc_mesh(),
        compiler_params=pltpu.CompilerParams(needs_layout_passes=False),
        scratch_types=[
            pltpu.VMEM((EPW,), jnp.int32),
            pltpu.VMEM((EPW,), jnp.int32),
            pltpu.VMEM((EPW,), jnp.float32),
            pltpu.VMEM((N,), jnp.float32),
            pltpu.VMEM((EPW,), jnp.float32),
        ],
    )
    def k(row_h, col_h, ew_h, dinv_h, out_h, row_v, col_v, ew_v, dinv_v, ws_v):
        c = lax.axis_index("c")
        s = lax.axis_index("s")
        base = (c * NS + s) * EPW
        pltpu.sync_copy(row_h.at[pl.ds(base, EPW)], row_v)
        pltpu.sync_copy(col_h.at[pl.ds(base, EPW)], col_v)
        pltpu.sync_copy(ew_h.at[pl.ds(base, EPW)], ew_v)
        pltpu.sync_copy(dinv_h, dinv_v)

        def step(i, _):
            o = i * 16
            r16 = row_v[pl.ds(o, 16)]
            c16 = col_v[pl.ds(o, 16)]
            w16 = ew_v[pl.ds(o, 16)]
            dr = plsc.load_gather(dinv_v, [r16])
            dc = plsc.load_gather(dinv_v, [c16])
            ws_v[pl.ds(o, 16)] = -(dr * w16 * dc)
            return _

        lax.fori_loop(0, EPW // 16, step, 0)
        pltpu.sync_copy(ws_v, out_h.at[pl.ds(base, EPW)])

    return k(row, col, ew, dinv)


NBUF = 5
NK = NCHT // NBUF
PRE = 3


def _cheb_layer(src2, row, ws, col2):
    @functools.partial(
        pl.kernel,
        out_type=(jax.ShapeDtypeStruct((NC * N, FH), jnp.float32),
                  jax.ShapeDtypeStruct((NC * N, FH), jnp.float32)),
        mesh=_sc_mesh(),
        compiler_params=pltpu.CompilerParams(use_tc_tiling_on_sc=False,
                                             needs_layout_passes=False),
        scratch_types=[
            pltpu.VMEM_SHARED((N, FH), jnp.float32),
            pltpu.VMEM_SHARED((N, FH), jnp.float32),
            pltpu.VMEM((EPT,), jnp.int32),
            pltpu.VMEM((ZR, FH), jnp.float32),
        ] + [pltpu.VMEM((CH, FH), jnp.float32) for _ in range(NBUF)]
          + [pltpu.VMEM((CH,), jnp.int32) for _ in range(NBUF)]
          + [pltpu.VMEM((CH,), jnp.float32) for _ in range(NBUF)]
          + [pltpu.SemaphoreType.DMA for _ in range(2 * NBUF)],
    )
    def k(src_h, row_h, ws_h, col_h, p1_h, p2_h,
          acc1, acc2, col_v, zbuf, *rest):
        bufs = rest[:NBUF]
        rstg = rest[NBUF:2 * NBUF]
        wstg = rest[2 * NBUF:3 * NBUF]
        gsem = rest[3 * NBUF:4 * NBUF]
        ssem = rest[4 * NBUF:]
        c = lax.axis_index("c")
        s = lax.axis_index("s")
        base = s * EPT
        pltpu.sync_copy(col_h.at[pl.ds(c * E + base, EPT)], col_v)

        z16 = jnp.zeros((16,), jnp.float32)
        for i in range(ZR):
            for j in range(FH // 16):
                zbuf[i, pl.ds(j * 16, 16)] = z16
        for a in (acc1, acc2):
            for q in range(NZ):
                pltpu.async_copy(zbuf, a.at[pl.ds(s * RPT + q * ZR, ZR)],
                                 rest[-1])

            @pl.when(s == NS - 1)
            def _(a=a):
                pltpu.async_copy(zbuf.at[pl.ds(0, TAIL)],
                                 a.at[pl.ds(TAIL_OFF, TAIL)], rest[-1])

        for a in (acc1, acc2):
            for q in range(NZ):
                pltpu.make_async_copy(zbuf, a.at[pl.ds(s * RPT + q * ZR, ZR)],
                                      rest[-1]).wait()

            @pl.when(s == NS - 1)
            def _(a=a):
                pltpu.make_async_copy(zbuf.at[pl.ds(0, TAIL)],
                                      a.at[pl.ds(TAIL_OFF, TAIL)],
                                      rest[-1]).wait()

        plsc.subcore_barrier()

        def writeout(a, out_h):
            def stg(q):
                return bufs[q % 2].at[pl.ds(0, ZR)]

            for q in range(NZ):
                if q >= 2:
                    pltpu.make_async_copy(
                        stg(q), out_h.at[pl.ds(0, ZR)], gsem[q % 2]).wait()
                pltpu.sync_copy(a.at[pl.ds(s * RPT + q * ZR, ZR)], stg(q))
                pltpu.async_copy(stg(q),
                                 out_h.at[pl.ds(c * N + s * RPT + q * ZR, ZR)],
                                 gsem[q % 2])
            for q in (NZ - 2, NZ - 1):
                pltpu.make_async_copy(
                    stg(q), out_h.at[pl.ds(0, ZR)], gsem[q % 2]).wait()

            @pl.when(s == NS - 1)
            def _():
                pltpu.sync_copy(a.at[pl.ds(TAIL_OFF, TAIL)],
                                zbuf.at[pl.ds(0, TAIL)])
                pltpu.sync_copy(zbuf.at[pl.ds(0, TAIL)],
                                out_h.at[pl.ds(c * N + TAIL_OFF, TAIL)])

        def hop(gather_ref, acc):
            def gissue(ch, b):
                pltpu.async_copy(gather_ref.at[col_v.at[pl.ds(ch * CH, CH)]],
                                 bufs[b], gsem[b])
                pltpu.async_copy(row_h.at[pl.ds(base + ch * CH, CH)],
                                 rstg[b], gsem[b])
                pltpu.async_copy(ws_h.at[pl.ds(base + ch * CH, CH)],
                                 wstg[b], gsem[b])

            def gdrain(b):
                pltpu.make_async_copy(src_h.at[pl.ds(0, CH)],
                                      bufs[b], gsem[b]).wait()
                pltpu.make_async_copy(row_h.at[pl.ds(0, CH)],
                                      rstg[b], gsem[b]).wait()
                pltpu.make_async_copy(row_h.at[pl.ds(0, CH)],
                                      wstg[b], gsem[b]).wait()

            def sdrain(b):
                pltpu.make_async_copy(src_h.at[pl.ds(0, CH)],
                                      bufs[b], ssem[b]).wait()

            for b in range(PRE):
                gissue(b, b)

            def round_(k_, car):
                for j in range(NBUF):
                    ch = k_ * NBUF + j
                    gdrain(j)
                    for g in range(CH // 16):
                        wvec = wstg[j][pl.ds(g * 16, 16)]
                        for i in range(16):
                            w = wvec[i]
                            r = g * 16 + i
                            for jj in range(FH // 16):
                                bufs[j][r, pl.ds(jj * 16, 16)] = (
                                    bufs[j][r, pl.ds(jj * 16, 16)] * w)
                    pltpu.async_copy(bufs[j], acc.at[rstg[j]], ssem[j],
                                     add=True)
                    tb = (j + PRE) % NBUF
                    if j < NBUF - PRE:
                        @pl.when(k_ >= 1)
                        def _w(tb=tb):
                            sdrain(tb)

                        gissue(ch + PRE, tb)
                    else:
                        @pl.when(k_ < NK - 1)
                        def _wg(tb=tb, ch=ch):
                            sdrain(tb)
                            gissue(ch + PRE, tb)

                return car

            lax.fori_loop(0, NK, round_, 0)
            for b in range(NBUF):
                sdrain(b)
            plsc.subcore_barrier()

        hop(src_h, acc1)
        writeout(acc1, p1_h)
        plsc.subcore_barrier()
        hop(p1_h, acc2)
        writeout(acc2, p2_h)

    return k(src2, row, ws, col2)


BN = 400


def _split(x):
    def body(x_ref, o_ref):
        o_ref[0] = x_ref[:, :FH]
        o_ref[1] = x_ref[:, FH:]

    return pl.pallas_call(
        body,
        grid=(N // BN,),
        in_specs=[pl.BlockSpec((BN, F), lambda i: (i, 0))],
        out_specs=pl.BlockSpec((2, BN, FH), lambda i: (0, i, 0)),
        out_shape=jax.ShapeDtypeStruct((2, N, FH), jnp.float32),
    )(x)


def _mix(src, t1, p2, W, b, relu, split_out):
    def body(s0, s1, t0, t1r, p0, p1, w_ref, b_ref, o_ref):
        xb = jnp.concatenate([s0[0], s1[0]], axis=1)
        tb = jnp.concatenate([t0[0], t1r[0]], axis=1)
        lb = jnp.concatenate([p0[0], p1[0]], axis=1)
        acc = jnp.dot(xb, w_ref[0] - w_ref[2], preferred_element_type=jnp.float32)
        acc += jnp.dot(tb, w_ref[1], preferred_element_type=jnp.float32)
        acc += jnp.dot(lb, w_ref[2] * 2.0, preferred_element_type=jnp.float32)
        acc += b_ref[...]
        if relu:
            acc = jnp.maximum(acc, 0.0)
        if split_out:
            o_ref[0] = acc[:, :FH]
            o_ref[1] = acc[:, FH:]
        else:
            o_ref[...] = acc

    half = lambda h: pl.BlockSpec((1, BN, FH), lambda i, _h=h: (_h, i, 0))
    if split_out:
        out_spec = pl.BlockSpec((2, BN, FH), lambda i: (0, i, 0))
        out_shape = jax.ShapeDtypeStruct((2, N, FH), jnp.float32)
    else:
        out_spec = pl.BlockSpec((BN, F), lambda i: (i, 0))
        out_shape = jax.ShapeDtypeStruct((N, F), jnp.float32)
    return pl.pallas_call(
        body,
        grid=(N // BN,),
        in_specs=[half(0), half(1), half(0), half(1), half(0), half(1),
                  pl.BlockSpec((3, F, F), lambda i: (0, 0, 0)),
                  pl.BlockSpec((1, F), lambda i: (0, 0))],
        out_specs=out_spec,
        out_shape=out_shape,
    )(src, src, t1, t1, p2, p2, W, b.reshape(1, F))


def _fc_head(h, fc1_w, fc1_b, fc2_w, fc2_b, fc3_w, fc3_b):
    def body(h_ref, w1_ref, b1_ref, w2_ref, b2_ref, w3_ref, b3_ref, o_ref):
        g = jnp.dot(h_ref[...], w1_ref[...], preferred_element_type=jnp.float32)
        g += b1_ref[...]
        g = jnp.dot(g, w2_ref[...], preferred_element_type=jnp.float32)
        g += b2_ref[...]
        g = jnp.dot(g, w3_ref[...], preferred_element_type=jnp.float32)
        g += b3_ref[...]
        o_ref[...] = g

    n_cls = fc3_w.shape[1]
    return pl.pallas_call(
        body,
        out_shape=jax.ShapeDtypeStruct((RES, n_cls), jnp.float32),
    )(h, fc1_w, fc1_b.reshape(1, -1), fc2_w, fc2_b.reshape(1, -1),
      fc3_w, fc3_b.reshape(1, -1))


def kernel(x, edge_index, edge_weight, W1, b1, W2, b2,
           fc1_w, fc1_b, fc2_w, fc2_b, fc3_w, fc3_b):
    row = edge_index[0]
    col = edge_index[1]
    row3 = row.reshape(NW, EPW // CH, CH)
    col2 = jnp.concatenate([col, col + N])

    deg = _deg_parts(row3, edge_weight).reshape(NC, N).sum(axis=0)
    dinv = jnp.where(deg > 0, jax.lax.rsqrt(jnp.where(deg > 0, deg, 1.0)), 0.0)
    ws = _w_scaled(row, col, edge_weight, dinv)

    def layer(src_split, W, b, relu, split_out):
        p1, p2 = _cheb_layer(src_split.reshape(NC * N, FH), row, ws, col2)
        return _mix(src_split, p1.reshape(NC, N, FH), p2.reshape(NC, N, FH),
                    W, b, relu, split_out)

    xh = _split(x)
    h = layer(xh, W1, b1, True, True)
    h2 = layer(h, W2, b2, False, False)
    return _fc_head(h2.reshape(RES, RES * F),
                    fc1_w, fc1_b, fc2_w, fc2_b, fc3_w, fc3_b)
